# baseline (speedup 1.0000x reference)
import jax
import jax.numpy as jnp
from jax.experimental import pallas as pl

M = 8
L = 8192
HALF = L // 2
NPAIR = HALF + 1 + 7
R = 6
BIG = 2 * L


def _topk_body(u_ref, v_ref, p_ref):
    u = u_ref[...]
    v = v_ref[...]
    rows = jax.lax.broadcasted_iota(jnp.int32, u.shape, 0)
    w = jnp.maximum(u, v)
    ks, uvals, vvals = [], [], []
    for _ in range(R):
        wmax = jnp.max(w, axis=0, keepdims=True)
        k = jnp.min(jnp.where(w == wmax, rows, BIG), axis=0)
        eqrow = rows == k[None, :]
        uvals.append(jnp.max(jnp.where(eqrow, u, -1.0), axis=0))
        vvals.append(jnp.max(jnp.where(eqrow, v, -1.0), axis=0))
        ks.append(k)
        w = jnp.where(eqrow, -1.0, w)
    vals, idxs = [], []
    for k, uv, vv in zip(ks, uvals, vvals):
        prim = uv >= vv
        single = (k == 0) | (k == HALF)
        vals.append(jnp.maximum(uv, vv))
        idxs.append(jnp.where(prim, k, L - k))
        vals.append(jnp.where(single, -1.0, jnp.minimum(uv, vv)))
        idxs.append(jnp.where(single, BIG, jnp.where(prim, L - k, k)))
    cv = jnp.stack(vals)
    ci = jnp.stack(idxs)
    for m in range(M):
        cmax = jnp.max(cv, axis=0, keepdims=True)
        sel = jnp.min(jnp.where(cv == cmax, ci, BIG), axis=0)
        f = (sel + 1).astype(jnp.float32)
        p_ref[m, :] = jnp.ceil(jnp.float32(L) / f).astype(jnp.int32)
        cv = jnp.where((cv == cmax) & (ci == sel[None, :]), -2.0, cv)


def kernel(x_input):
    b, length, d = x_input.shape
    x_DFT = jnp.fft.fft(x_input, axis=1)
    a = jnp.abs(x_DFT)
    a_t = jnp.transpose(a, (1, 0, 2)).reshape(length, b * d)
    n_cols = b * d
    neg1 = jnp.full((1, n_cols), -1.0, jnp.float32)
    u = jnp.concatenate([a_t[: HALF + 1], jnp.tile(neg1, (7, 1))], axis=0)
    v = jnp.concatenate(
        [neg1, jnp.flip(a_t[HALF + 1 :], axis=0), jnp.tile(neg1, (8, 1))],
        axis=0,
    )
    p = pl.pallas_call(
        _topk_body,
        grid=(n_cols // 128,),
        in_specs=[
            pl.BlockSpec((NPAIR, 128), lambda j: (0, j)),
            pl.BlockSpec((NPAIR, 128), lambda j: (0, j)),
        ],
        out_specs=pl.BlockSpec((M, 128), lambda j: (0, j)),
        out_shape=jax.ShapeDtypeStruct((M, n_cols), jnp.int32),
    )(u, v)
    p = jnp.transpose(p.reshape(M, b, d), (1, 0, 2))
    return p.astype(jnp.int64)


# speedup vs baseline: 1.1868x; 1.1868x over previous
import jax
import jax.numpy as jnp
from jax.experimental import pallas as pl

M = 8
L = 8192
HALF = L // 2
R = 6
BIG = 2 * L


def _topk_body(u_ref, v_ref, p_ref):
    u = u_ref[...]
    v = v_ref[...]
    rows = jax.lax.broadcasted_iota(jnp.int32, u.shape, 0)
    w = jnp.maximum(u, v)
    ks, uvals, vvals = [], [], []
    for _ in range(R):
        wmax = jnp.max(w, axis=0, keepdims=True)
        k = jnp.min(jnp.where(w == wmax, rows, BIG), axis=0)
        eqrow = rows == k[None, :]
        uvals.append(jnp.max(jnp.where(eqrow, u, -1.0), axis=0))
        vvals.append(jnp.max(jnp.where(eqrow, v, -1.0), axis=0))
        ks.append(k)
        w = jnp.where(eqrow, -1.0, w)
    vals, idxs = [], []
    for k, uv, vv in zip(ks, uvals, vvals):
        prim = uv >= vv
        midx = jnp.where(k == 0, HALF, L - k)
        vals.append(jnp.maximum(uv, vv))
        idxs.append(jnp.where(prim, k, midx))
        vals.append(jnp.minimum(uv, vv))
        idxs.append(jnp.where(prim, midx, k))
    cv = jnp.stack(vals)
    ci = jnp.stack(idxs)
    for m in range(M):
        cmax = jnp.max(cv, axis=0, keepdims=True)
        sel = jnp.min(jnp.where(cv == cmax, ci, BIG), axis=0)
        f = (sel + 1).astype(jnp.float32)
        p_ref[m, :] = jnp.ceil(jnp.float32(L) / f).astype(jnp.int32)
        cv = jnp.where((cv == cmax) & (ci == sel[None, :]), -2.0, cv)


def kernel(x_input):
    b, length, d = x_input.shape
    x_DFT = jnp.fft.fft(x_input, axis=1)
    a = jnp.abs(x_DFT)
    n_cols = b * d
    u_nat = a[:, :HALF, :]
    v_nat = jnp.concatenate(
        [a[:, HALF : HALF + 1, :], jnp.flip(a[:, HALF + 1 :, :], axis=1)],
        axis=1,
    )
    u = jnp.transpose(u_nat, (1, 0, 2)).reshape(HALF, n_cols)
    v = jnp.transpose(v_nat, (1, 0, 2)).reshape(HALF, n_cols)
    p = pl.pallas_call(
        _topk_body,
        grid=(n_cols // 128,),
        in_specs=[
            pl.BlockSpec((HALF, 128), lambda j: (0, j)),
            pl.BlockSpec((HALF, 128), lambda j: (0, j)),
        ],
        out_specs=pl.BlockSpec((M, 128), lambda j: (0, j)),
        out_shape=jax.ShapeDtypeStruct((M, n_cols), jnp.int32),
    )(u, v)
    p = jnp.transpose(p.reshape(M, b, d), (1, 0, 2))
    return p.astype(jnp.int64)


# speedup vs baseline: 1.3046x; 1.0993x over previous
import jax
import jax.numpy as jnp
from jax.experimental import pallas as pl

M = 8
L = 8192
HALF = L // 2
R = 6
BIG = 2 * L


def _topk_body(a_ref, p_ref):
    lo = a_ref[:HALF, :]
    hi = a_ref[HALF:, :]
    rows = jax.lax.broadcasted_iota(jnp.int32, lo.shape, 0)
    vals, idxs = [], []
    for side, base in ((lo, 0), (hi, HALF)):
        w = side
        for _ in range(R):
            wmax = jnp.max(w, axis=0, keepdims=True)
            k = jnp.min(jnp.where(w == wmax, rows, BIG), axis=0)
            vals.append(wmax[0])
            idxs.append(k + base)
            w = jnp.where(rows == k[None, :], -1.0, w)
    cv = jnp.stack(vals)
    ci = jnp.stack(idxs)
    for m in range(M):
        cmax = jnp.max(cv, axis=0, keepdims=True)
        sel = jnp.min(jnp.where(cv == cmax, ci, BIG), axis=0)
        f = (sel + 1).astype(jnp.float32)
        p_ref[m, :] = jnp.ceil(jnp.float32(L) / f).astype(jnp.int32)
        cv = jnp.where((cv == cmax) & (ci == sel[None, :]), -2.0, cv)


def kernel(x_input):
    b, length, d = x_input.shape
    x_DFT = jnp.fft.fft(x_input, axis=1)
    a = jnp.abs(x_DFT)
    a_t = jnp.transpose(a, (1, 0, 2)).reshape(length, b * d)
    n_cols = b * d
    p = pl.pallas_call(
        _topk_body,
        grid=(n_cols // 128,),
        in_specs=[pl.BlockSpec((length, 128), lambda j: (0, j))],
        out_specs=pl.BlockSpec((M, 128), lambda j: (0, j)),
        out_shape=jax.ShapeDtypeStruct((M, n_cols), jnp.int32),
    )(a_t)
    p = jnp.transpose(p.reshape(M, b, d), (1, 0, 2))
    return p.astype(jnp.int64)


# speedup vs baseline: 1.3272x; 1.0173x over previous
import jax
import jax.numpy as jnp
from jax.experimental import pallas as pl

M = 8
L = 8192
HALF = L // 2
R = 5
BIG = 2 * L


def _topk_body(a_ref, p_ref):
    lo = a_ref[:HALF, :]
    hi = a_ref[HALF:, :]
    rows = jax.lax.broadcasted_iota(jnp.int32, lo.shape, 0)
    vals, idxs = [], []
    for side, base in ((lo, 0), (hi, HALF)):
        w = side
        for _ in range(R):
            wmax = jnp.max(w, axis=0, keepdims=True)
            k = jnp.min(jnp.where(w == wmax, rows, BIG), axis=0)
            vals.append(wmax[0])
            idxs.append(k + base)
            w = jnp.where(rows == k[None, :], -1.0, w)
    cv = jnp.stack(vals)
    ci = jnp.stack(idxs)
    for m in range(M):
        cmax = jnp.max(cv, axis=0, keepdims=True)
        sel = jnp.min(jnp.where(cv == cmax, ci, BIG), axis=0)
        f = (sel + 1).astype(jnp.float32)
        p_ref[m, :] = jnp.ceil(jnp.float32(L) / f).astype(jnp.int32)
        cv = jnp.where((cv == cmax) & (ci == sel[None, :]), -2.0, cv)


def kernel(x_input):
    b, length, d = x_input.shape
    x_DFT = jnp.fft.fft(x_input, axis=1)
    a = jnp.abs(x_DFT)
    a_t = jnp.transpose(a, (1, 0, 2)).reshape(length, b * d)
    n_cols = b * d
    p = pl.pallas_call(
        _topk_body,
        grid=(n_cols // 128,),
        in_specs=[pl.BlockSpec((length, 128), lambda j: (0, j))],
        out_specs=pl.BlockSpec((M, 128), lambda j: (0, j)),
        out_shape=jax.ShapeDtypeStruct((M, n_cols), jnp.int32),
    )(a_t)
    p = jnp.transpose(p.reshape(M, b, d), (1, 0, 2))
    return p.astype(jnp.int64)
